# baseline (speedup 1.0000x reference)
import dataclasses
import functools

import jax
import jax.numpy as jnp
from jax import lax
from jax.experimental import pallas as pl
from jax.experimental.pallas import tpu as pltpu
from jax.experimental.pallas import tpu_sc as plsc

_NC = 2
_NS = 16
_NW = _NC * _NS
_CHUNK = 128
_CW = 128
_ZR = 128
_NPAD = 10240
_PADNODE = 10200


def _vmesh():
    return plsc.VectorSubcoreMesh(core_axis_name="c", subcore_axis_name="s",
                                  num_cores=_NC, num_subcores=_NS)


def _sc_params():
    cp = pltpu.CompilerParams()
    if "needs_layout_passes" in pltpu.CompilerParams.__dataclass_fields__:
        cp = dataclasses.replace(cp, needs_layout_passes=False)
    return cp


def _sc_degrees(src1d, dst1d):
    ept = src1d.shape[0] // _NW
    stripe = _NPAD // _NS

    @functools.partial(
        pl.kernel,
        out_type=jax.ShapeDtypeStruct((_NC, 2, _NPAD), jnp.float32),
        mesh=_vmesh(),
        scratch_types=[
            pltpu.VMEM((ept,), jnp.int32),
            pltpu.VMEM((ept,), jnp.int32),
            pltpu.VMEM((_NPAD,), jnp.float32),
            pltpu.VMEM((_NPAD,), jnp.float32),
            pltpu.VMEM((stripe,), jnp.float32),
            pltpu.VMEM((stripe,), jnp.float32),
            pltpu.VMEM_SHARED((_NS, 2, _NPAD), jnp.float32),
        ],
        compiler_params=_sc_params(),
    )
    def deg(src_hbm, dst_hbm, out_hbm, sidx, didx, hs, hd, tmp, accb, stage):
        c = lax.axis_index("c")
        s = lax.axis_index("s")
        wid = c * _NS + s
        zeros16 = jnp.zeros((16,), jnp.float32)
        ones16 = jnp.ones((16,), jnp.float32)

        pre = pl.multiple_of(wid * ept, 128)
        pltpu.sync_copy(src_hbm.at[pl.ds(pre, ept)], sidx)
        pltpu.sync_copy(dst_hbm.at[pl.ds(pre, ept)], didx)

        @pl.loop(0, _NPAD, step=16)
        def _(i):
            hs[pl.ds(i, 16)] = zeros16
            hd[pl.ds(i, 16)] = zeros16

        @pl.loop(0, ept, step=16)
        def _(l):
            plsc.addupdate_scatter(hs, [sidx[pl.ds(l, 16)]], ones16)
            plsc.addupdate_scatter(hd, [didx[pl.ds(l, 16)]], ones16)

        pltpu.sync_copy(hs, stage.at[s, 0])
        pltpu.sync_copy(hd, stage.at[s, 1])
        plsc.subcore_barrier()

        @pl.loop(0, 2)
        def _(k):
            @pl.loop(0, stripe, step=16)
            def _(i):
                accb[pl.ds(i, 16)] = zeros16

            @pl.loop(0, _NS)
            def _(t):
                pltpu.sync_copy(stage.at[t, k, pl.ds(pl.multiple_of(s * stripe, 128), stripe)], tmp)

                @pl.loop(0, stripe, step=16)
                def _(i):
                    accb[pl.ds(i, 16)] = accb[pl.ds(i, 16)] + tmp[pl.ds(i, 16)]

            pltpu.sync_copy(accb, out_hbm.at[c, k, pl.ds(pl.multiple_of(s * stripe, 128), stripe)])

    return deg(src1d, dst1d)


def _sc_aggregate(g, src2d, dst2d):
    n_chunks = src2d.shape[0]
    cpt = n_chunks // _NW
    ept = cpt * _CHUNK
    rows_per_tile = _NPAD // _NS

    @functools.partial(
        pl.kernel,
        out_type=jax.ShapeDtypeStruct((_NC, _NPAD, _CW), jnp.float32),
        mesh=_vmesh(),
        scratch_types=[
            pltpu.VMEM((_CHUNK,), jnp.int32),
            pltpu.VMEM((_CHUNK,), jnp.int32),
            pltpu.VMEM((_CHUNK, _CW), jnp.float32),
            pltpu.VMEM((_ZR, _CW), jnp.float32),
            pltpu.VMEM_SHARED((_NPAD, _CW), jnp.float32),
            pltpu.SemaphoreType.DMA,
        ],
    )
    def agg(g_hbm, src_hbm, dst_hbm, out_hbm,
            sidx, didx, rows0, zbuf, acc, gsem):
        c = lax.axis_index("c")
        s = lax.axis_index("s")
        wid = c * _NS + s
        base_chunk = wid * cpt
        zrow = jnp.zeros((1, 16), jnp.float32)

        @pl.loop(0, _ZR)
        def _(r):
            @pl.loop(0, _CW, step=16)
            def _(l):
                zbuf.at[pl.ds(r, 1), pl.ds(l, 16)][...] = zrow

        @pl.loop(0, rows_per_tile, step=_ZR)
        def _(r0):
            pltpu.sync_copy(zbuf, acc.at[pl.ds(pl.multiple_of(s * rows_per_tile + r0, 8), _ZR)])

        plsc.subcore_barrier()

        @pl.loop(0, cpt)
        def _(j):
            pltpu.sync_copy(src_hbm.at[base_chunk + j], sidx)
            pltpu.sync_copy(dst_hbm.at[base_chunk + j], didx)
            pltpu.async_copy(g_hbm.at[sidx], rows0, gsem).wait()
            pltpu.sync_copy(rows0, acc.at[didx], add=True)

        plsc.subcore_barrier()

        @pl.loop(0, rows_per_tile, step=_ZR)
        def _(r0):
            base = pl.multiple_of(s * rows_per_tile + r0, 8)
            pltpu.sync_copy(acc.at[pl.ds(base, _ZR)], zbuf)
            pltpu.sync_copy(zbuf, out_hbm.at[c, pl.ds(base, _ZR)])

    return agg(g, src2d, dst2d)


def _dot(a, b):
    return jnp.dot(a, b, preferred_element_type=jnp.float32,
                   precision=lax.Precision.HIGHEST)


def _tc_prelayer(x, cnts):
    n, d = x.shape

    def body(x_ref, cnt_ref, g_ref, dor_ref, dir_ref):
        cs = cnt_ref[0, 0, :, :] + cnt_ref[1, 0, :, :]
        cd = cnt_ref[0, 1, :, :] + cnt_ref[1, 1, :, :]
        dor = lax.rsqrt(jnp.maximum(cs, 1.0))[:n]
        dir_ = lax.rsqrt(jnp.maximum(cd, 1.0))[:n]
        dor_ref[...] = dor
        dir_ref[...] = dir_
        g_ref[pl.ds(0, n), :] = x_ref[...] * dor

    return pl.pallas_call(
        body,
        out_shape=(
            jax.ShapeDtypeStruct((_NPAD, d), jnp.float32),
            jax.ShapeDtypeStruct((n, 1), jnp.float32),
            jax.ShapeDtypeStruct((n, 1), jnp.float32),
        ),
    )(x, cnts.reshape(_NC, 2, _NPAD, 1))


def _tc_layer1(a1, dir_, W1, b1, dor, W2s):
    n = dir_.shape[0]
    br = 1000
    grid = (n // br,)
    d_in = W1.shape[0]
    h1 = W1.shape[1]
    nchunk = W2s.shape[0]

    def body(a_ref, dir_ref, w1_ref, b1_ref, dor_ref, w2_ref, *outs):
        a = (a_ref[0] + a_ref[1]) * dir_ref[...]
        h = jnp.maximum(_dot(a, w1_ref[...]) + b1_ref[...], 0.0)
        hs = h * dor_ref[...]
        for c in range(nchunk):
            outs[c][...] = _dot(hs, w2_ref[c])

    return pl.pallas_call(
        body,
        grid=grid,
        in_specs=[
            pl.BlockSpec((_NC, br, d_in), lambda i: (0, i, 0)),
            pl.BlockSpec((br, 1), lambda i: (i, 0)),
            pl.BlockSpec((d_in, h1), lambda i: (0, 0)),
            pl.BlockSpec((1, h1), lambda i: (0, 0)),
            pl.BlockSpec((br, 1), lambda i: (i, 0)),
            pl.BlockSpec(W2s.shape, lambda i: (0, 0, 0)),
        ],
        out_specs=[pl.BlockSpec((br, _CW), lambda i: (i, 0))] * nchunk,
        out_shape=[jax.ShapeDtypeStruct((_NPAD, _CW), jnp.float32)] * nchunk,
    )(a1, dir_, W1, b1.reshape(1, h1), dor, W2s)


def _tc_midlayer(parts, dir_, b, dor, Ws):
    n = dir_.shape[0]
    br = 1000
    grid = (n // br,)
    nchunk = Ws.shape[0]
    hwid = Ws.shape[1]

    def body(*refs):
        a_refs = refs[:nchunk]
        dir_ref, b_ref, dor_ref, w_ref = refs[nchunk:nchunk + 4]
        outs = refs[nchunk + 4:]
        agg = jnp.concatenate([r[0] + r[1] for r in a_refs], axis=1)
        h = jnp.maximum(agg * dir_ref[...] + b_ref[...], 0.0)
        hs = h * dor_ref[...]
        for c in range(nchunk):
            outs[c][...] = _dot(hs, w_ref[c])

    return pl.pallas_call(
        body,
        grid=grid,
        in_specs=(
            [pl.BlockSpec((_NC, br, _CW), lambda i: (0, i, 0))] * nchunk
            + [
                pl.BlockSpec((br, 1), lambda i: (i, 0)),
                pl.BlockSpec((1, hwid), lambda i: (0, 0)),
                pl.BlockSpec((br, 1), lambda i: (i, 0)),
                pl.BlockSpec(Ws.shape, lambda i: (0, 0, 0)),
            ]
        ),
        out_specs=[pl.BlockSpec((br, _CW), lambda i: (i, 0))] * nchunk,
        out_shape=[jax.ShapeDtypeStruct((_NPAD, _CW), jnp.float32)] * nchunk,
    )(*parts, dir_, b.reshape(1, hwid), dor, Ws)


def _tc_lastlayer(parts, dir_, b):
    n = dir_.shape[0]
    br = 1000
    grid = (n // br,)
    nchunk = len(parts)
    hwid = b.shape[0]

    def body(*refs):
        a_refs = refs[:nchunk]
        dir_ref, b_ref = refs[nchunk:nchunk + 2]
        out_ref = refs[nchunk + 2]
        agg = jnp.concatenate([r[0] + r[1] for r in a_refs], axis=1)
        out_ref[...] = jnp.maximum(agg * dir_ref[...] + b_ref[...], 0.0)

    return pl.pallas_call(
        body,
        grid=grid,
        in_specs=(
            [pl.BlockSpec((_NC, br, _CW), lambda i: (0, i, 0))] * nchunk
            + [
                pl.BlockSpec((br, 1), lambda i: (i, 0)),
                pl.BlockSpec((1, hwid), lambda i: (0, 0)),
            ]
        ),
        out_specs=pl.BlockSpec((br, hwid), lambda i: (i, 0)),
        out_shape=jax.ShapeDtypeStruct((n, hwid), jnp.float32),
    )(*parts, dir_, b.reshape(1, hwid))


def kernel(in_feat, edge_index, W1, b1, W2, b2, W3, b3):
    n, d_in = in_feat.shape
    e = edge_index.shape[1]
    h1 = W1.shape[1]
    h2 = W2.shape[1]
    h3 = W3.shape[1]

    epc = _CHUNK * _NW
    e_pad = -(-e // epc) * epc
    src = edge_index[0]
    dst = edge_index[1]
    if e_pad != e:
        fill = jnp.full((e_pad - e,), _PADNODE, dtype=jnp.int32)
        src = jnp.concatenate([src, fill])
        dst = jnp.concatenate([dst, fill])

    W2s = W2.reshape(W2.shape[0], h2 // _CW, _CW).transpose(1, 0, 2)
    W3s = W3.reshape(W3.shape[0], h3 // _CW, _CW).transpose(1, 0, 2)

    src2d = src.reshape(e_pad // _CHUNK, _CHUNK)
    dst2d = dst.reshape(e_pad // _CHUNK, _CHUNK)

    cnts = _sc_degrees(src, dst)
    g1, dor, dir_ = _tc_prelayer(in_feat, cnts)

    a1 = _sc_aggregate(g1, src2d, dst2d)
    g2 = _tc_layer1(a1, dir_, W1, b1, dor, W2s)

    a2 = [_sc_aggregate(gc, src2d, dst2d) for gc in g2]
    g3 = _tc_midlayer(a2, dir_, b2, dor, W3s)

    a3 = [_sc_aggregate(gc, src2d, dst2d) for gc in g3]
    return _tc_lastlayer(a3, dir_, b3)


# speedup vs baseline: 1.7465x; 1.7465x over previous
import dataclasses
import functools

import jax
import jax.numpy as jnp
import numpy as np
from jax import lax
from jax.experimental import pallas as pl
from jax.experimental.pallas import tpu as pltpu
from jax.experimental.pallas import tpu_sc as plsc

_NC = 2
_NS = 16
_NW = _NC * _NS
_CHUNK = 128
_CW = 128
_ZR = 128
_NPAD = 10240
_PADNODE = 10200


def _vmesh():
    return plsc.VectorSubcoreMesh(core_axis_name="c", subcore_axis_name="s",
                                  num_cores=_NC, num_subcores=_NS)


def _sc_params():
    cp = pltpu.CompilerParams()
    if "needs_layout_passes" in pltpu.CompilerParams.__dataclass_fields__:
        cp = dataclasses.replace(cp, needs_layout_passes=False)
    return cp


def _sc_degrees(src1d, dst1d):
    ept = src1d.shape[0] // _NW
    stripe = _NPAD // _NS

    @functools.partial(
        pl.kernel,
        out_type=jax.ShapeDtypeStruct((_NC, 2, _NPAD), jnp.float32),
        mesh=_vmesh(),
        scratch_types=[
            pltpu.VMEM((ept,), jnp.int32),
            pltpu.VMEM((ept,), jnp.int32),
            pltpu.VMEM((_NPAD,), jnp.float32),
            pltpu.VMEM((_NPAD,), jnp.float32),
            pltpu.VMEM((stripe,), jnp.float32),
            pltpu.VMEM((stripe,), jnp.float32),
            pltpu.VMEM_SHARED((_NS, 2, _NPAD), jnp.float32),
        ],
        compiler_params=_sc_params(),
    )
    def deg(src_hbm, dst_hbm, out_hbm, sidx, didx, hs, hd, tmp, accb, stage):
        c = lax.axis_index("c")
        s = lax.axis_index("s")
        wid = c * _NS + s
        zeros16 = jnp.zeros((16,), jnp.float32)
        ones16 = jnp.ones((16,), jnp.float32)

        pre = pl.multiple_of(wid * ept, 128)
        pltpu.sync_copy(src_hbm.at[pl.ds(pre, ept)], sidx)
        pltpu.sync_copy(dst_hbm.at[pl.ds(pre, ept)], didx)

        @pl.loop(0, _NPAD, step=16)
        def _(i):
            hs[pl.ds(i, 16)] = zeros16
            hd[pl.ds(i, 16)] = zeros16

        @pl.loop(0, ept, step=16)
        def _(l):
            plsc.addupdate_scatter(hs, [sidx[pl.ds(l, 16)]], ones16)
            plsc.addupdate_scatter(hd, [didx[pl.ds(l, 16)]], ones16)

        pltpu.sync_copy(hs, stage.at[s, 0])
        pltpu.sync_copy(hd, stage.at[s, 1])
        plsc.subcore_barrier()

        @pl.loop(0, 2)
        def _(k):
            @pl.loop(0, stripe, step=16)
            def _(i):
                accb[pl.ds(i, 16)] = zeros16

            @pl.loop(0, _NS)
            def _(t):
                pltpu.sync_copy(stage.at[t, k, pl.ds(pl.multiple_of(s * stripe, 128), stripe)], tmp)

                @pl.loop(0, stripe, step=16)
                def _(i):
                    accb[pl.ds(i, 16)] = accb[pl.ds(i, 16)] + tmp[pl.ds(i, 16)]

            pltpu.sync_copy(accb, out_hbm.at[c, k, pl.ds(pl.multiple_of(s * stripe, 128), stripe)])

    return deg(src1d, dst1d)


def _sc_aggregate(g, src2d, dst2d):
    n_chunks = src2d.shape[0]
    cpt = n_chunks // _NW
    ept = cpt * _CHUNK
    rows_per_tile = _NPAD // _NS

    @functools.partial(
        pl.kernel,
        out_type=jax.ShapeDtypeStruct((_NC, _NPAD, _CW), jnp.float32),
        mesh=_vmesh(),
        scratch_types=[
            pltpu.VMEM((ept,), jnp.int32),
            pltpu.VMEM((_CHUNK,), jnp.int32),
            pltpu.VMEM((_CHUNK, _CW), jnp.float32),
            pltpu.VMEM((_ZR, _CW), jnp.float32),
            pltpu.VMEM_SHARED((_NPAD, _CW), jnp.float32),
            pltpu.SemaphoreType.DMA,
        ],
    )
    def agg(g_hbm, srcf_hbm, dst_hbm, out_hbm,
            sidx, didx, rows0, zbuf, acc, gsem):
        c = lax.axis_index("c")
        s = lax.axis_index("s")
        wid = c * _NS + s
        base_chunk = wid * cpt
        zrow = jnp.zeros((1, 16), jnp.float32)

        pltpu.sync_copy(
            srcf_hbm.at[pl.ds(pl.multiple_of(wid * ept, 128), ept)], sidx)

        @pl.loop(0, _ZR)
        def _(r):
            @pl.loop(0, _CW, step=16)
            def _(l):
                zbuf.at[pl.ds(r, 1), pl.ds(l, 16)][...] = zrow

        @pl.loop(0, rows_per_tile, step=_ZR)
        def _(r0):
            pltpu.sync_copy(zbuf, acc.at[pl.ds(pl.multiple_of(s * rows_per_tile + r0, 8), _ZR)])

        plsc.subcore_barrier()

        @pl.loop(0, cpt)
        def _(j):
            pltpu.sync_copy(dst_hbm.at[base_chunk + j], didx)
            gi = sidx.at[pl.ds(pl.multiple_of(j * _CHUNK, 128), _CHUNK)]
            pltpu.async_copy(g_hbm.at[gi], rows0, gsem).wait()
            pltpu.sync_copy(rows0, acc.at[didx], add=True)

        plsc.subcore_barrier()

        @pl.loop(0, rows_per_tile, step=_ZR)
        def _(r0):
            base = pl.multiple_of(s * rows_per_tile + r0, 8)
            pltpu.sync_copy(acc.at[pl.ds(base, _ZR)], zbuf)
            pltpu.sync_copy(zbuf, out_hbm.at[c, pl.ds(base, _ZR)])

    return agg(g, src2d.reshape(-1), dst2d)


def _dot(a, b):
    return jnp.dot(a, b, preferred_element_type=jnp.float32,
                   precision=lax.Precision.HIGHEST)


def _tc_prelayer(x, cnts):
    n, d = x.shape

    def body(x_ref, cnt_ref, g_ref, dor_ref, dir_ref):
        cs = cnt_ref[0, 0, :, :] + cnt_ref[1, 0, :, :]
        cd = cnt_ref[0, 1, :, :] + cnt_ref[1, 1, :, :]
        dor = lax.rsqrt(jnp.maximum(cs, 1.0))[:n]
        dir_ = lax.rsqrt(jnp.maximum(cd, 1.0))[:n]
        dor_ref[...] = dor
        dir_ref[...] = dir_
        g_ref[pl.ds(0, n), :] = x_ref[...] * dor

    return pl.pallas_call(
        body,
        out_shape=(
            jax.ShapeDtypeStruct((_NPAD, d), jnp.float32),
            jax.ShapeDtypeStruct((n, 1), jnp.float32),
            jax.ShapeDtypeStruct((n, 1), jnp.float32),
        ),
    )(x, cnts.reshape(_NC, 2, _NPAD, 1))


def _tc_layer1(a1, dir_, W1, b1, dor, W2s):
    n = dir_.shape[0]
    br = 1000
    grid = (n // br,)
    d_in = W1.shape[0]
    h1 = W1.shape[1]
    nchunk = W2s.shape[0]

    def body(a_ref, dir_ref, w1_ref, b1_ref, dor_ref, w2_ref, *outs):
        a = (a_ref[0] + a_ref[1]) * dir_ref[...]
        h = jnp.maximum(_dot(a, w1_ref[...]) + b1_ref[...], 0.0)
        hs = h * dor_ref[...]
        for c in range(nchunk):
            outs[c][...] = _dot(hs, w2_ref[c])

    return pl.pallas_call(
        body,
        grid=grid,
        in_specs=[
            pl.BlockSpec((_NC, br, d_in), lambda i: (0, i, 0)),
            pl.BlockSpec((br, 1), lambda i: (i, 0)),
            pl.BlockSpec((d_in, h1), lambda i: (0, 0)),
            pl.BlockSpec((1, h1), lambda i: (0, 0)),
            pl.BlockSpec((br, 1), lambda i: (i, 0)),
            pl.BlockSpec(W2s.shape, lambda i: (0, 0, 0)),
        ],
        out_specs=[pl.BlockSpec((br, _CW), lambda i: (i, 0))] * nchunk,
        out_shape=[jax.ShapeDtypeStruct((_NPAD, _CW), jnp.float32)] * nchunk,
    )(a1, dir_, W1, b1.reshape(1, h1), dor, W2s)


def _tc_midlayer(parts, dir_, b, dor, Ws):
    n = dir_.shape[0]
    br = 1000
    grid = (n // br,)
    nchunk = Ws.shape[0]
    hwid = Ws.shape[1]

    def body(*refs):
        a_refs = refs[:nchunk]
        dir_ref, b_ref, dor_ref, w_ref = refs[nchunk:nchunk + 4]
        outs = refs[nchunk + 4:]
        agg = jnp.concatenate([r[0] + r[1] for r in a_refs], axis=1)
        h = jnp.maximum(agg * dir_ref[...] + b_ref[...], 0.0)
        hs = h * dor_ref[...]
        for c in range(nchunk):
            outs[c][...] = _dot(hs, w_ref[c])

    return pl.pallas_call(
        body,
        grid=grid,
        in_specs=(
            [pl.BlockSpec((_NC, br, _CW), lambda i: (0, i, 0))] * nchunk
            + [
                pl.BlockSpec((br, 1), lambda i: (i, 0)),
                pl.BlockSpec((1, hwid), lambda i: (0, 0)),
                pl.BlockSpec((br, 1), lambda i: (i, 0)),
                pl.BlockSpec(Ws.shape, lambda i: (0, 0, 0)),
            ]
        ),
        out_specs=[pl.BlockSpec((br, _CW), lambda i: (i, 0))] * nchunk,
        out_shape=[jax.ShapeDtypeStruct((_NPAD, _CW), jnp.float32)] * nchunk,
    )(*parts, dir_, b.reshape(1, hwid), dor, Ws)


def _tc_lastlayer(parts, dir_, b):
    n = dir_.shape[0]
    br = 1000
    grid = (n // br,)
    nchunk = len(parts)
    hwid = b.shape[0]

    def body(*refs):
        a_refs = refs[:nchunk]
        dir_ref, b_ref = refs[nchunk:nchunk + 2]
        out_ref = refs[nchunk + 2]
        agg = jnp.concatenate([r[0] + r[1] for r in a_refs], axis=1)
        out_ref[...] = jnp.maximum(agg * dir_ref[...] + b_ref[...], 0.0)

    return pl.pallas_call(
        body,
        grid=grid,
        in_specs=(
            [pl.BlockSpec((_NC, br, _CW), lambda i: (0, i, 0))] * nchunk
            + [
                pl.BlockSpec((br, 1), lambda i: (i, 0)),
                pl.BlockSpec((1, hwid), lambda i: (0, 0)),
            ]
        ),
        out_specs=pl.BlockSpec((br, hwid), lambda i: (i, 0)),
        out_shape=jax.ShapeDtypeStruct((n, hwid), jnp.float32),
    )(*parts, dir_, b.reshape(1, hwid))


def kernel(in_feat, edge_index, W1, b1, W2, b2, W3, b3):
    n, d_in = in_feat.shape
    e = edge_index.shape[1]
    h1 = W1.shape[1]
    h2 = W2.shape[1]
    h3 = W3.shape[1]

    epc = _CHUNK * _NW
    e_pad = -(-e // epc) * epc
    src = edge_index[0]
    dst = edge_index[1]
    if e_pad != e:
        fill = (n + np.arange(e_pad - e, dtype=np.int32) % (_NPAD - n)
                ).astype(np.int32)
        src = jnp.concatenate([src, jnp.asarray(fill)])
        dst = jnp.concatenate([dst, jnp.asarray(fill)])
    n_real = e // _CHUNK
    n_tot = e_pad // _CHUNK
    cpt = n_tot // _NW
    extra = n_real - (n_real // _NW) * _NW
    perm = []
    r, dmy = 0, n_real
    for w in range(_NW):
        take = n_real // _NW + (1 if w < extra else 0)
        perm.extend(range(r, r + take))
        r += take
        perm.extend(range(dmy, dmy + cpt - take))
        dmy += cpt - take
    perm = np.asarray(perm, dtype=np.int32)

    W2s = W2.reshape(W2.shape[0], h2 // _CW, _CW).transpose(1, 0, 2)
    W3s = W3.reshape(W3.shape[0], h3 // _CW, _CW).transpose(1, 0, 2)

    src2d = src.reshape(n_tot, _CHUNK)[perm]
    dst2d = dst.reshape(n_tot, _CHUNK)[perm]
    src = src2d.reshape(-1)
    dst = dst2d.reshape(-1)

    cnts = _sc_degrees(src, dst)
    g1, dor, dir_ = _tc_prelayer(in_feat, cnts)

    a1 = _sc_aggregate(g1, src2d, dst2d)
    g2 = _tc_layer1(a1, dir_, W1, b1, dor, W2s)

    a2 = [_sc_aggregate(gc, src2d, dst2d) for gc in g2]
    g3 = _tc_midlayer(a2, dir_, b2, dor, W3s)

    a3 = [_sc_aggregate(gc, src2d, dst2d) for gc in g3]
    return _tc_lastlayer(a3, dir_, b3)
